# baseline (speedup 1.0000x reference)
import jax
import jax.numpy as jnp
from jax.experimental import pallas as pl
from jax.experimental.pallas import tpu as pltpu

_N = 2048
_L = 125
_LP = 128
_RS = 8
_CS = 256
_INF = 1e18
_NEG = -1e30


def _mst_body(dist_ref, parent_ref, order_ref, edged_ref):
    row_i = jax.lax.broadcasted_iota(jnp.int32, (_RS, _CS), 0)
    col_i = jax.lax.broadcasted_iota(jnp.int32, (_RS, _CS), 1)
    iota2 = row_i * _CS + col_i

    iota2f = iota2.astype(jnp.float32)

    key0 = jnp.where(iota2 == 0, _INF, dist_ref[0])
    zf = jnp.zeros((_RS, _CS), jnp.float32)

    cidx0 = jax.lax.broadcasted_iota(jnp.int32, (_LP, 1), 0).astype(jnp.float32)
    cidx1 = cidx0 + jnp.float32(_LP)
    ones_col = jnp.ones((_LP, 1), jnp.float32)
    riota = jax.lax.broadcasted_iota(jnp.int32, (_RS, 1), 0).astype(jnp.float32)

    def body(k, st):
        key, parent, order, edged = st
        m_s = jnp.min(key)
        eqf = jnp.where(key == m_s, jnp.float32(1.0), jnp.float32(0.0))
        sidx = (jax.lax.dot(eqf[:, :_LP], cidx0,
                            preferred_element_type=jnp.float32)
                + jax.lax.dot(eqf[:, _LP:], cidx1,
                              preferred_element_type=jnp.float32))
        scnt = (jax.lax.dot(eqf[:, :_LP], ones_col,
                            preferred_element_type=jnp.float32)
                + jax.lax.dot(eqf[:, _LP:], ones_col,
                              preferred_element_type=jnp.float32))
        rw = scnt * riota
        for sh in (4, 2, 1):
            sidx = sidx + pltpu.roll(sidx, sh, 0)
            scnt = scnt + pltpu.roll(scnt, sh, 0)
            rw = rw + pltpu.roll(rw, sh, 0)
        u_vec = sidx + jnp.float32(_CS) * rw
        u_fast = u_vec[0, 0]
        cnt = scnt[0, 0]
        u_f = jax.lax.cond(
            cnt == 1.0,
            lambda: u_fast,
            lambda: jnp.min(jnp.where(key == m_s, iota2f, jnp.float32(_N))))
        u = u_f.astype(jnp.int32)
        du = dist_ref[u]
        order = jnp.where(iota2 == k, u_f, order)
        eq_u = iota2f == u_f
        edged = jnp.where(eq_u, key, edged)
        key = jnp.where(eq_u, _INF, key)
        upd = (du < key) & (key < _INF)
        parent = jnp.where(upd, u_f, parent)
        key = jnp.where(upd, du, key)
        return key, parent, order, edged

    _, parent, order, edged = jax.lax.fori_loop(
        1, _N, body, (key0, zf, zf, zf))
    parent_ref[...] = parent.astype(jnp.int32)
    order_ref[...] = order.astype(jnp.int32)
    edged_ref[...] = edged


def _viterbi_body(order_ref, parent_ref, w_ref, unary_ref, a_ref,
                  out_ref, score_ref, cand_ref, oh_ref):
    score_ref[...] = unary_ref[...] * 1.5
    a_mat = a_ref[...]
    lane = jax.lax.broadcasted_iota(jnp.int32, (1, _LP), 1)
    row_idx = jax.lax.broadcasted_iota(jnp.int32, (_LP, _LP), 0).astype(jnp.float32)

    def up(i, carry):
        k = _N - i
        c = order_ref[k]
        p = parent_ref[c]
        w = w_ref[c]
        sc = score_ref[pl.ds(c, 1), :]
        sc_col = jnp.transpose(sc)
        x = sc_col + w * a_mat
        m = jnp.max(x, axis=0, keepdims=True)
        cand_ref[pl.ds(c, 1), :] = jnp.min(
            jnp.where(x == m, row_idx, float(_LP)), axis=0, keepdims=True)
        score_ref[pl.ds(p, 1), :] = score_ref[pl.ds(p, 1), :] + m
        return carry

    jax.lax.fori_loop(1, _N, up, 0)

    root = order_ref[0]
    sroot = score_ref[pl.ds(root, 1), :]
    mx = jnp.max(sroot)
    lroot = jnp.min(jnp.where(sroot == mx, lane, _LP))
    oh_root = jnp.where(lane == lroot, jnp.float32(1.0), jnp.float32(0.0))
    oh_ref[pl.ds(root, 1), :] = oh_root
    out_ref[pl.ds(root, 1), :] = jnp.where(
        oh_root > 0.5, jnp.float32(100000.0), jnp.float32(-100000.0))

    lane_f = lane.astype(jnp.float32)

    def down(k, carry):
        c = order_ref[k]
        p = parent_ref[c]
        cand_row = cand_ref[pl.ds(c, 1), :]
        ohp = oh_ref[pl.ds(p, 1), :]
        v = ohp * cand_row
        s_v = jnp.max(v, axis=(0, 1), keepdims=True)
        oh = jnp.where(lane_f == s_v, jnp.float32(1.0), jnp.float32(0.0))
        oh_ref[pl.ds(c, 1), :] = oh
        out_ref[pl.ds(c, 1), :] = jnp.where(
            oh > 0.5, jnp.float32(100000.0), jnp.float32(-100000.0))
        return carry

    jax.lax.fori_loop(1, _N, down, 0)


def kernel(unary, dist, A):
    dist3 = dist.reshape(_N, _RS, _CS)
    parent, order, edged = pl.pallas_call(
        _mst_body,
        out_shape=(
            jax.ShapeDtypeStruct((_RS, _CS), jnp.int32),
            jax.ShapeDtypeStruct((_RS, _CS), jnp.int32),
            jax.ShapeDtypeStruct((_RS, _CS), jnp.float32),
        ),
        in_specs=[pl.BlockSpec(memory_space=pltpu.VMEM)],
        out_specs=(
            pl.BlockSpec(memory_space=pltpu.VMEM),
            pl.BlockSpec(memory_space=pltpu.VMEM),
            pl.BlockSpec(memory_space=pltpu.VMEM),
        ),
        compiler_params=pltpu.CompilerParams(
            vmem_limit_bytes=100 * 1024 * 1024),
    )(dist3)

    parent = parent.reshape(_N)
    order = order.reshape(_N)
    w = jnp.exp(-edged.reshape(_N) / 2.0)

    unary_p = jnp.pad(unary, ((0, 0), (0, _LP - _L)), constant_values=_NEG)
    a_p = jnp.pad(A, ((0, _LP - _L), (0, _LP - _L)))

    out = pl.pallas_call(
        _viterbi_body,
        out_shape=jax.ShapeDtypeStruct((_N, _LP), jnp.float32),
        in_specs=[
            pl.BlockSpec(memory_space=pltpu.SMEM),
            pl.BlockSpec(memory_space=pltpu.SMEM),
            pl.BlockSpec(memory_space=pltpu.SMEM),
            pl.BlockSpec(memory_space=pltpu.VMEM),
            pl.BlockSpec(memory_space=pltpu.VMEM),
        ],
        out_specs=pl.BlockSpec(memory_space=pltpu.VMEM),
        scratch_shapes=[
            pltpu.VMEM((_N, _LP), jnp.float32),
            pltpu.VMEM((_N, _LP), jnp.float32),
            pltpu.VMEM((_N, _LP), jnp.float32),
        ],
    )(order, parent, w, unary_p, a_p)
    return out[:, :_L]


# speedup vs baseline: 1.1041x; 1.1041x over previous
import jax
import jax.numpy as jnp
from jax.experimental import pallas as pl
from jax.experimental.pallas import tpu as pltpu

_N = 2048
_L = 125
_LP = 128
_RS = 8
_CS = 256
_INF = 1e18
_NEG = -1e30


def _mst_body(dist_ref, parent_ref, order_ref, edged_ref):
    row_i = jax.lax.broadcasted_iota(jnp.int32, (_RS, _CS), 0)
    col_i = jax.lax.broadcasted_iota(jnp.int32, (_RS, _CS), 1)
    iota2 = row_i * _CS + col_i

    iota2f = iota2.astype(jnp.float32)

    key0 = jnp.where(iota2 == 0, _INF, dist_ref[0])
    zf = jnp.zeros((_RS, _CS), jnp.float32)

    def body(k, st):
        key, parent, order, edged = st
        m_s = jnp.min(key)
        u_f = jnp.min(jnp.where(key == m_s, iota2f, jnp.float32(_N)))
        u = u_f.astype(jnp.int32)
        du = dist_ref[u]
        order = jnp.where(iota2 == k, u_f, order)
        eq_u = iota2f == u_f
        edged = jnp.where(eq_u, key, edged)
        key = jnp.where(eq_u, _INF, key)
        upd = (du < key) & (key < _INF)
        parent = jnp.where(upd, u_f, parent)
        key = jnp.where(upd, du, key)
        return key, parent, order, edged

    _, parent, order, edged = jax.lax.fori_loop(
        1, _N, body, (key0, zf, zf, zf))
    parent_ref[...] = parent.astype(jnp.int32)
    order_ref[...] = order.astype(jnp.int32)
    edged_ref[...] = edged


def _viterbi_body(order_ref, parent_ref, w_ref, unary_ref, a_ref,
                  out_ref, score_ref, cand_ref, oh_ref):
    score_ref[...] = unary_ref[...] * 1.5
    a_mat = a_ref[...]
    lane = jax.lax.broadcasted_iota(jnp.int32, (1, _LP), 1)
    row_idx = jax.lax.broadcasted_iota(jnp.int32, (_LP, _LP), 0).astype(jnp.float32)

    def up(i, carry):
        k = _N - i
        c = order_ref[k]
        p = parent_ref[c]
        w = w_ref[c]
        sc = score_ref[pl.ds(c, 1), :]
        sc_col = jnp.transpose(sc)
        x = sc_col + w * a_mat
        m = jnp.max(x, axis=0, keepdims=True)
        cand_ref[pl.ds(c, 1), :] = jnp.min(
            jnp.where(x == m, row_idx, float(_LP)), axis=0, keepdims=True)
        score_ref[pl.ds(p, 1), :] = score_ref[pl.ds(p, 1), :] + m
        return carry

    jax.lax.fori_loop(1, _N, up, 0)

    root = order_ref[0]
    sroot = score_ref[pl.ds(root, 1), :]
    mx = jnp.max(sroot)
    lroot = jnp.min(jnp.where(sroot == mx, lane, _LP))
    oh_root = jnp.where(lane == lroot, jnp.float32(1.0), jnp.float32(0.0))
    oh_ref[pl.ds(root, 1), :] = oh_root
    out_ref[pl.ds(root, 1), :] = jnp.where(
        oh_root > 0.5, jnp.float32(100000.0), jnp.float32(-100000.0))

    lane_f = lane.astype(jnp.float32)

    def down(k, carry):
        c = order_ref[k]
        p = parent_ref[c]
        cand_row = cand_ref[pl.ds(c, 1), :]
        ohp = oh_ref[pl.ds(p, 1), :]
        v = ohp * cand_row
        s_v = jnp.max(v, axis=(0, 1), keepdims=True)
        oh = jnp.where(lane_f == s_v, jnp.float32(1.0), jnp.float32(0.0))
        oh_ref[pl.ds(c, 1), :] = oh
        out_ref[pl.ds(c, 1), :] = jnp.where(
            oh > 0.5, jnp.float32(100000.0), jnp.float32(-100000.0))
        return carry

    jax.lax.fori_loop(1, _N, down, 0)


def kernel(unary, dist, A):
    dist3 = dist.reshape(_N, _RS, _CS)
    parent, order, edged = pl.pallas_call(
        _mst_body,
        out_shape=(
            jax.ShapeDtypeStruct((_RS, _CS), jnp.int32),
            jax.ShapeDtypeStruct((_RS, _CS), jnp.int32),
            jax.ShapeDtypeStruct((_RS, _CS), jnp.float32),
        ),
        in_specs=[pl.BlockSpec(memory_space=pltpu.VMEM)],
        out_specs=(
            pl.BlockSpec(memory_space=pltpu.VMEM),
            pl.BlockSpec(memory_space=pltpu.VMEM),
            pl.BlockSpec(memory_space=pltpu.VMEM),
        ),
        compiler_params=pltpu.CompilerParams(
            vmem_limit_bytes=100 * 1024 * 1024),
    )(dist3)

    parent = parent.reshape(_N)
    order = order.reshape(_N)
    w = jnp.exp(-edged.reshape(_N) / 2.0)

    unary_p = jnp.pad(unary, ((0, 0), (0, _LP - _L)), constant_values=_NEG)
    a_p = jnp.pad(A, ((0, _LP - _L), (0, _LP - _L)))

    out = pl.pallas_call(
        _viterbi_body,
        out_shape=jax.ShapeDtypeStruct((_N, _LP), jnp.float32),
        in_specs=[
            pl.BlockSpec(memory_space=pltpu.SMEM),
            pl.BlockSpec(memory_space=pltpu.SMEM),
            pl.BlockSpec(memory_space=pltpu.SMEM),
            pl.BlockSpec(memory_space=pltpu.VMEM),
            pl.BlockSpec(memory_space=pltpu.VMEM),
        ],
        out_specs=pl.BlockSpec(memory_space=pltpu.VMEM),
        scratch_shapes=[
            pltpu.VMEM((_N, _LP), jnp.float32),
            pltpu.VMEM((_N, _LP), jnp.float32),
            pltpu.VMEM((_N, _LP), jnp.float32),
        ],
    )(order, parent, w, unary_p, a_p)
    return out[:, :_L]
